# baseline (speedup 1.0000x reference)
import functools

import jax
import jax.numpy as jnp
from jax import lax
from jax.experimental import pallas as pl
from jax.experimental.pallas import tpu as pltpu
from jax.experimental.pallas import tpu_sc as plsc

N = 10000
E = 320000
D_IN = 128
D_HID = 64
D_OUT = 40

NC = 2
NS = 16
NW = NC * NS
EPW = E // NW
CB = 80
NCHUNK = EPW // CB
NBUF = 5
L = 16
STRIPE = 624
LAST_BASE = STRIPE * (NS - 1)
LAST_ROWS = N - LAST_BASE

_MESH = plsc.VectorSubcoreMesh(
    core_axis_name="c", subcore_axis_name="s", num_cores=NC, num_subcores=NS)
_SC_PARAMS = pltpu.CompilerParams(
    needs_layout_passes=False, use_tc_tiling_on_sc=False)


HR = 80


@functools.partial(
    pl.kernel,
    out_type=jax.ShapeDtypeStruct((NW, HR, 128), jnp.float32),
    mesh=_MESH,
    compiler_params=_SC_PARAMS,
    scratch_types=[
        pltpu.VMEM((NCHUNK, CB), jnp.int32),
        pltpu.VMEM((HR, 128), jnp.float32),
    ],
)
def _sc_degree(edges_hbm, out_hbm, colv, hist):
    cid = lax.axis_index("c")
    sid = lax.axis_index("s")
    wid = sid * NC + cid
    pltpu.sync_copy(edges_hbm.at[1].at[wid], colv)

    zeros = jnp.zeros((L,), jnp.float32)

    def zero_body(r, _):
        for c in range(128 // L):
            hist[r, pl.ds(c * L, L)] = zeros
        return 0

    lax.fori_loop(0, HR, zero_body, 0, unroll=4)

    ones = jnp.ones((L,), jnp.float32)

    def add_body(r, _):
        for c in range(CB // L):
            idx = colv[r, pl.ds(c * L, L)]
            plsc.addupdate_scatter(
                hist, [lax.shift_right_logical(idx, 7), idx & 127], ones)
        return 0

    lax.fori_loop(0, NCHUNK, add_body, 0, unroll=2)
    pltpu.sync_copy(hist, out_hbm.at[wid])


def _sc_spmm_body(D, h_hbm, edges_hbm, z_hbm, out_hbm,
                  rowi, coli, gbuf, acc, gsem):
    cid = lax.axis_index("c")
    sid = lax.axis_index("s")
    wid = sid * NC + cid
    pltpu.sync_copy(edges_hbm.at[0].at[wid], rowi)
    pltpu.sync_copy(edges_hbm.at[1].at[wid], coli)

    def _gather(j, b):
        return pltpu.make_async_copy(h_hbm.at[rowi.at[j]], gbuf.at[b],
                                     gsem.at[b])

    for b in range(NBUF):
        _gather(b, b).start()

    @pl.when(sid < NS - 1)
    def _():
        pltpu.sync_copy(z_hbm.at[pl.ds(0, STRIPE)],
                        acc.at[pl.ds(sid * STRIPE, STRIPE)])

    @pl.when(sid == NS - 1)
    def _():
        pltpu.sync_copy(z_hbm, acc.at[pl.ds(LAST_BASE, LAST_ROWS)])

    plsc.subcore_barrier()

    def chunk_body(g, _):
        for b in range(NBUF):
            j = g * NBUF + b
            _gather(j, b).wait()
            pltpu.sync_copy(gbuf.at[b], acc.at[coli.at[j]], add=True)

            @pl.when(j + NBUF < NCHUNK)
            def _():
                _gather(j + NBUF, b).start()
        return 0

    lax.fori_loop(0, NCHUNK // NBUF, chunk_body, 0)
    plsc.subcore_barrier()

    @pl.when(sid < NS - 1)
    def _():
        pltpu.sync_copy(acc.at[pl.ds(sid * STRIPE, STRIPE)],
                        out_hbm.at[cid, pl.ds(sid * STRIPE, STRIPE)])

    @pl.when(sid == NS - 1)
    def _():
        pltpu.sync_copy(acc.at[pl.ds(LAST_BASE, LAST_ROWS)],
                        out_hbm.at[cid, pl.ds(LAST_BASE, LAST_ROWS)])


def _make_sc_spmm(D):
    return functools.partial(
        pl.kernel,
        out_type=jax.ShapeDtypeStruct((NC, N, D), jnp.float32),
        mesh=_MESH,
        compiler_params=_SC_PARAMS,
        scratch_types=[
            pltpu.VMEM((NCHUNK, CB), jnp.int32),
            pltpu.VMEM((NCHUNK, CB), jnp.int32),
            pltpu.VMEM((NBUF, CB, D), jnp.float32),
            pltpu.VMEM_SHARED((N, D), jnp.float32),
            pltpu.SemaphoreType.DMA((NBUF,)),
        ],
    )(functools.partial(_sc_spmm_body, D))


_sc_spmm64 = _make_sc_spmm(D_HID)
_sc_spmm40 = _make_sc_spmm(D_OUT)


def _dinv_from_hist(hist):
    deg = jnp.sum(hist, axis=0).reshape(HR * 128)[:N] + 1.0
    return lax.rsqrt(deg)[:, None]


def _tc1a_body(x_ref, w1_ref, out_ref):
    out_ref[...] = jnp.dot(x_ref[...], w1_ref[...],
                           preferred_element_type=jnp.float32)


def _pack2(y):
    y3 = y.reshape(N // 2, 2, y.shape[1])
    return jnp.concatenate([y3[:, 0, :], y3[:, 1, :]], axis=1)


def _unpack2(y, d):
    a, b = y[:, :d], y[:, d:]
    return jnp.stack([a, b], axis=1).reshape(2 * y.shape[0], d)


def _tc1b_body(hist_ref, h_ref, out_ref):
    h1p = h_ref[...] * _dinv_from_hist(hist_ref[...])
    out_ref[...] = _pack2(h1p)


def _tc1c_body(hist_ref, d64_ref, d40_ref):
    dinv = _dinv_from_hist(hist_ref[...])
    d64_ref[...] = _pack2(jnp.broadcast_to(dinv, (N, D_HID)))
    d40_ref[...] = _pack2(jnp.broadcast_to(dinv, (N, D_OUT)))


def _tc2_body(p_ref, h1p_ref, d64_ref, w2blk_ref, b1_ref, out_ref):
    aggP = p_ref[0] + p_ref[1] + h1p_ref[...]
    hP = jnp.maximum(aggP * d64_ref[...] + b1_ref[...], 0.0)
    h2P = jnp.dot(hP, w2blk_ref[...], preferred_element_type=jnp.float32)
    out_ref[...] = h2P * _pack2_view40(d64_ref[...])


def _pack2_view40(d64):
    return jnp.concatenate([d64[:, :D_OUT], d64[:, D_HID:D_HID + D_OUT]],
                           axis=1)


def _tc3_body(p_ref, h2p_ref, d40_ref, b2_ref, out_ref):
    zP = (p_ref[0] + p_ref[1] + h2p_ref[...]) * d40_ref[...] + b2_ref[...]

    def _lsm(z):
        m = jnp.max(z, axis=1, keepdims=True)
        e = jnp.exp(z - m)
        s = jnp.sum(e, axis=1, keepdims=True)
        return z - m - jnp.log(s)

    za = _lsm(zP[:, :D_OUT])
    zb = _lsm(zP[:, D_OUT:])
    out_ref[...] = _unpack2(jnp.concatenate([za, zb], axis=1), D_OUT)


def _tc_call(body, n_in, out_shape):
    multi = isinstance(out_shape, tuple)
    return pl.pallas_call(
        body,
        out_shape=out_shape,
        in_specs=[pl.BlockSpec(memory_space=pltpu.VMEM)] * n_in,
        out_specs=(tuple(pl.BlockSpec(memory_space=pltpu.VMEM)
                         for _ in out_shape)
                   if multi else pl.BlockSpec(memory_space=pltpu.VMEM)),
    )


def kernel(x, edge_index, W1, b1, W2, b2):
    edges4 = edge_index.astype(jnp.int32).reshape(2, NW, NCHUNK, CB)

    z64 = jnp.zeros((LAST_ROWS, D_HID), jnp.float32)
    z40 = jnp.zeros((LAST_ROWS, D_OUT), jnp.float32)

    hist = _sc_degree(edges4)

    h1 = _tc_call(_tc1a_body, 2,
                  jax.ShapeDtypeStruct((N, D_HID), jnp.float32))(x, W1)
    h1p = _tc_call(_tc1b_body, 2,
                   jax.ShapeDtypeStruct((N // 2, 2 * D_HID), jnp.float32))(
        hist, h1)

    d64, d40 = _tc_call(
        _tc1c_body, 1,
        (jax.ShapeDtypeStruct((N // 2, 2 * D_HID), jnp.float32),
         jax.ShapeDtypeStruct((N // 2, 2 * D_OUT), jnp.float32)))(hist)

    p1 = _sc_spmm64(h1p.reshape(N, D_HID), edges4, z64)

    w2blk = jnp.zeros((2 * D_HID, 2 * D_OUT), jnp.float32)
    w2blk = w2blk.at[:D_HID, :D_OUT].set(W2)
    w2blk = w2blk.at[D_HID:, D_OUT:].set(W2)

    PB = 1000
    h2p = pl.pallas_call(
        _tc2_body,
        grid=(N // 2 // PB,),
        out_shape=jax.ShapeDtypeStruct((N // 2, 2 * D_OUT), jnp.float32),
        in_specs=[
            pl.BlockSpec((NC, PB, 2 * D_HID), lambda i: (0, i, 0)),
            pl.BlockSpec((PB, 2 * D_HID), lambda i: (i, 0)),
            pl.BlockSpec((PB, 2 * D_HID), lambda i: (i, 0)),
            pl.BlockSpec((2 * D_HID, 2 * D_OUT), lambda i: (0, 0)),
            pl.BlockSpec((1, 2 * D_HID), lambda i: (0, 0)),
        ],
        out_specs=pl.BlockSpec((PB, 2 * D_OUT), lambda i: (i, 0)),
    )(p1.reshape(NC, N // 2, 2 * D_HID), h1p, d64, w2blk,
      jnp.concatenate([b1, b1]).reshape(1, 2 * D_HID))

    p2 = _sc_spmm40(h2p.reshape(N, D_OUT), edges4, z40)

    out = pl.pallas_call(
        _tc3_body,
        grid=(N // 2 // PB,),
        out_shape=jax.ShapeDtypeStruct((N, D_OUT), jnp.float32),
        in_specs=[
            pl.BlockSpec((NC, PB, 2 * D_OUT), lambda i: (0, i, 0)),
            pl.BlockSpec((PB, 2 * D_OUT), lambda i: (i, 0)),
            pl.BlockSpec((PB, 2 * D_OUT), lambda i: (i, 0)),
            pl.BlockSpec((1, 2 * D_OUT), lambda i: (0, 0)),
        ],
        out_specs=pl.BlockSpec((2 * PB, D_OUT), lambda i: (i, 0)),
    )(p2.reshape(NC, N // 2, 2 * D_OUT), h2p, d40,
      jnp.concatenate([b2, b2]).reshape(1, 2 * D_OUT))
    return out


# speedup vs baseline: 1.0075x; 1.0075x over previous
import functools

import jax
import jax.numpy as jnp
from jax import lax
from jax.experimental import pallas as pl
from jax.experimental.pallas import tpu as pltpu
from jax.experimental.pallas import tpu_sc as plsc

N = 10000
E = 320000
D_IN = 128
D_HID = 64
D_OUT = 40

NC = 2
NS = 16
NW = NC * NS
EPW = E // NW
CB = 80
NCHUNK = EPW // CB
NBUF = 5
L = 16
STRIPE = 624
LAST_BASE = STRIPE * (NS - 1)
LAST_ROWS = N - LAST_BASE

_MESH = plsc.VectorSubcoreMesh(
    core_axis_name="c", subcore_axis_name="s", num_cores=NC, num_subcores=NS)
_SC_PARAMS = pltpu.CompilerParams(
    needs_layout_passes=False, use_tc_tiling_on_sc=False)


HR = 80


@functools.partial(
    pl.kernel,
    out_type=jax.ShapeDtypeStruct((NW, HR, 128), jnp.float32),
    mesh=_MESH,
    compiler_params=_SC_PARAMS,
    scratch_types=[
        pltpu.VMEM((NCHUNK, CB), jnp.int32),
        pltpu.VMEM((HR, 128), jnp.float32),
    ],
)
def _sc_degree(edges_hbm, out_hbm, colv, hist):
    cid = lax.axis_index("c")
    sid = lax.axis_index("s")
    wid = sid * NC + cid
    pltpu.sync_copy(edges_hbm.at[1].at[wid], colv)

    zeros = jnp.zeros((L,), jnp.float32)

    def zero_body(r, _):
        for c in range(128 // L):
            hist[r, pl.ds(c * L, L)] = zeros
        return 0

    lax.fori_loop(0, HR, zero_body, 0, unroll=4)

    ones = jnp.ones((L,), jnp.float32)

    def add_body(r, _):
        for c in range(CB // L):
            idx = colv[r, pl.ds(c * L, L)]
            plsc.addupdate_scatter(
                hist, [lax.shift_right_logical(idx, 7), idx & 127], ones)
        return 0

    lax.fori_loop(0, NCHUNK, add_body, 0, unroll=2)
    pltpu.sync_copy(hist, out_hbm.at[wid])


def _sc_spmm_body(D, h_hbm, edges_hbm, z_hbm, out_hbm,
                  rowi, coli, gbuf, acc, gsem):
    cid = lax.axis_index("c")
    sid = lax.axis_index("s")
    wid = sid * NC + cid
    pltpu.sync_copy(edges_hbm.at[0].at[wid], rowi)
    pltpu.sync_copy(edges_hbm.at[1].at[wid], coli)

    def _gather(j, b):
        return pltpu.make_async_copy(h_hbm.at[rowi.at[j]], gbuf.at[b],
                                     gsem.at[b])

    for b in range(NBUF):
        _gather(b, b).start()

    @pl.when(sid < NS - 1)
    def _():
        pltpu.sync_copy(z_hbm.at[pl.ds(0, STRIPE)],
                        acc.at[pl.ds(sid * STRIPE, STRIPE)])

    @pl.when(sid == NS - 1)
    def _():
        pltpu.sync_copy(z_hbm, acc.at[pl.ds(LAST_BASE, LAST_ROWS)])

    plsc.subcore_barrier()

    def chunk_body(g, _):
        for b in range(NBUF):
            j = g * NBUF + b
            _gather(j, b).wait()
            pltpu.sync_copy(gbuf.at[b], acc.at[coli.at[j]], add=True)

            @pl.when(j + NBUF < NCHUNK)
            def _():
                _gather(j + NBUF, b).start()
        return 0

    lax.fori_loop(0, NCHUNK // NBUF, chunk_body, 0)
    plsc.subcore_barrier()

    @pl.when(sid < NS - 1)
    def _():
        pltpu.sync_copy(acc.at[pl.ds(sid * STRIPE, STRIPE)],
                        out_hbm.at[cid, pl.ds(sid * STRIPE, STRIPE)])

    @pl.when(sid == NS - 1)
    def _():
        pltpu.sync_copy(acc.at[pl.ds(LAST_BASE, LAST_ROWS)],
                        out_hbm.at[cid, pl.ds(LAST_BASE, LAST_ROWS)])


def _make_sc_spmm(D):
    return functools.partial(
        pl.kernel,
        out_type=jax.ShapeDtypeStruct((NC, N, D), jnp.float32),
        mesh=_MESH,
        compiler_params=_SC_PARAMS,
        scratch_types=[
            pltpu.VMEM((NCHUNK, CB), jnp.int32),
            pltpu.VMEM((NCHUNK, CB), jnp.int32),
            pltpu.VMEM((NBUF, CB, D), jnp.float32),
            pltpu.VMEM_SHARED((N, D), jnp.float32),
            pltpu.SemaphoreType.DMA((NBUF,)),
        ],
    )(functools.partial(_sc_spmm_body, D))


_sc_spmm64 = _make_sc_spmm(D_HID)
_sc_spmm40 = _make_sc_spmm(D_OUT)


def _dinv_from_hist(hist):
    deg = jnp.sum(hist, axis=0).reshape(HR * 128)[:N] + 1.0
    return lax.rsqrt(deg)[:, None]


def _tc1a_body(x_ref, w1_ref, out_ref):
    out_ref[...] = jnp.dot(x_ref[...], w1_ref[...],
                           preferred_element_type=jnp.float32)


def _pack2(y):
    y3 = y.reshape(N // 2, 2, y.shape[1])
    return jnp.concatenate([y3[:, 0, :], y3[:, 1, :]], axis=1)


def _unpack2(y, d):
    a, b = y[:, :d], y[:, d:]
    return jnp.stack([a, b], axis=1).reshape(2 * y.shape[0], d)


def _tc1b_body(hist_ref, h_ref, out_ref):
    h1p = h_ref[...] * _dinv_from_hist(hist_ref[...])
    out_ref[...] = _pack2(h1p)


def _tc1c_body(hist_ref, d64_ref, d40_ref):
    dinv = _dinv_from_hist(hist_ref[...])
    d64_ref[...] = _pack2(jnp.broadcast_to(dinv, (N, D_HID)))
    d40_ref[...] = _pack2(jnp.broadcast_to(dinv, (N, D_OUT)))


def _tc2_body(p_ref, h1p_ref, d64_ref, w2blk_ref, b1_ref, out_ref):
    aggP = p_ref[0] + p_ref[1] + h1p_ref[...]
    hP = jnp.maximum(aggP * d64_ref[...] + b1_ref[...], 0.0)
    h2P = jnp.dot(hP, w2blk_ref[...], preferred_element_type=jnp.float32)
    out_ref[...] = h2P * _pack2_view40(d64_ref[...])


def _pack2_view40(d64):
    return jnp.concatenate([d64[:, :D_OUT], d64[:, D_HID:D_HID + D_OUT]],
                           axis=1)


def _tc3_body(p_ref, h2p_ref, d40_ref, b2_ref, out_ref):
    zP = (p_ref[0] + p_ref[1] + h2p_ref[...]) * d40_ref[...] + b2_ref[...]

    def _lsm(z):
        m = jnp.max(z, axis=1, keepdims=True)
        e = jnp.exp(z - m)
        s = jnp.sum(e, axis=1, keepdims=True)
        return z - m - jnp.log(s)

    za = _lsm(zP[:, :D_OUT])
    zb = _lsm(zP[:, D_OUT:])
    out_ref[...] = _unpack2(jnp.concatenate([za, zb], axis=1), D_OUT)


def _tc_call(body, n_in, out_shape):
    multi = isinstance(out_shape, tuple)
    return pl.pallas_call(
        body,
        out_shape=out_shape,
        in_specs=[pl.BlockSpec(memory_space=pltpu.VMEM)] * n_in,
        out_specs=(tuple(pl.BlockSpec(memory_space=pltpu.VMEM)
                         for _ in out_shape)
                   if multi else pl.BlockSpec(memory_space=pltpu.VMEM)),
    )


def kernel(x, edge_index, W1, b1, W2, b2):
    edges4 = edge_index.astype(jnp.int32).reshape(2, NW, NCHUNK, CB)

    z64 = jnp.zeros((LAST_ROWS, D_HID), jnp.float32)
    z40 = jnp.zeros((LAST_ROWS, D_OUT), jnp.float32)

    hist = _sc_degree(edges4)

    h1 = _tc_call(_tc1a_body, 2,
                  jax.ShapeDtypeStruct((N, D_HID), jnp.float32))(x, W1)
    h1p = _tc_call(_tc1b_body, 2,
                   jax.ShapeDtypeStruct((N // 2, 2 * D_HID), jnp.float32))(
        hist, h1)

    d64, d40 = _tc_call(
        _tc1c_body, 1,
        (jax.ShapeDtypeStruct((N // 2, 2 * D_HID), jnp.float32),
         jax.ShapeDtypeStruct((N // 2, 2 * D_OUT), jnp.float32)))(hist)

    p1 = _sc_spmm64(h1p.reshape(N, D_HID), edges4, z64)

    w2blk = jnp.zeros((2 * D_HID, 2 * D_OUT), jnp.float32)
    w2blk = w2blk.at[:D_HID, :D_OUT].set(W2)
    w2blk = w2blk.at[D_HID:, D_OUT:].set(W2)

    h2p = _tc_call(_tc2_body, 5,
                   jax.ShapeDtypeStruct((N // 2, 2 * D_OUT), jnp.float32))(
        p1.reshape(NC, N // 2, 2 * D_HID), h1p, d64, w2blk,
        jnp.concatenate([b1, b1]).reshape(1, 2 * D_HID))

    p2 = _sc_spmm40(h2p.reshape(N, D_OUT), edges4, z40)

    PB = 1000
    out = pl.pallas_call(
        _tc3_body,
        grid=(N // 2 // PB,),
        out_shape=jax.ShapeDtypeStruct((N, D_OUT), jnp.float32),
        in_specs=[
            pl.BlockSpec((NC, PB, 2 * D_OUT), lambda i: (0, i, 0)),
            pl.BlockSpec((PB, 2 * D_OUT), lambda i: (i, 0)),
            pl.BlockSpec((PB, 2 * D_OUT), lambda i: (i, 0)),
            pl.BlockSpec((1, 2 * D_OUT), lambda i: (0, 0)),
        ],
        out_specs=pl.BlockSpec((2 * PB, D_OUT), lambda i: (i, 0)),
    )(p2.reshape(NC, N // 2, 2 * D_OUT), h2p, d40,
      jnp.concatenate([b2, b2]).reshape(1, 2 * D_OUT))
    return out
